# baseline (speedup 1.0000x reference)
import functools

import jax
import jax.numpy as jnp
from jax import lax
from jax.experimental import pallas as pl
from jax.experimental.pallas import tpu as pltpu
from jax.experimental.pallas import tpu_sc as plsc

T = 2048
D = 2048
DFF = 1408
E = 8
K = 2

B = 256
NBLK = (T * K) // B + E - 1
NBLK_PAD = 32
NROWS = NBLK * B

_interp = False


def _plan_body(x_ref, gw_ref, v1_ref, v2_ref, d1_ref, d2_ref, plan_ref):
    x = x_ref[...]
    gw = gw_ref[...]
    logits = lax.dot_general(x, gw, (((1,), (1,)), ((), ())),
                             preferred_element_type=jnp.float32)
    m = jnp.max(logits, axis=1, keepdims=True)
    ex = jnp.exp(logits - m)
    p = ex / jnp.sum(ex, axis=1, keepdims=True)

    eio = lax.broadcasted_iota(jnp.int32, (T, E), 1)
    m1 = jnp.max(p, axis=1, keepdims=True)
    i1 = jnp.min(jnp.where(p >= m1, eio, E), axis=1, keepdims=True)
    p2 = jnp.where(eio == i1, -1.0, p)
    m2 = jnp.max(p2, axis=1, keepdims=True)
    i2 = jnp.min(jnp.where(p2 >= m2, eio, E), axis=1, keepdims=True)

    ind1 = (eio == i1).astype(jnp.float32)
    ind2 = (eio == i2).astype(jnp.float32)
    ind12 = ind1 + ind2

    CH = 128
    tri = (lax.broadcasted_iota(jnp.int32, (CH, CH), 0)
           > lax.broadcasted_iota(jnp.int32, (CH, CH), 1)).astype(jnp.float32)
    run = jnp.zeros((1, E), jnp.float32)
    c1_chunks = []
    for i in range(T // CH):
        blk = lax.slice(ind12, (i * CH, 0), ((i + 1) * CH, E))
        c1_chunks.append(lax.dot_general(tri, blk, (((1,), (0,)), ((), ())),
                                         precision=lax.Precision.HIGHEST,
                                         preferred_element_type=jnp.float32)
                         + run)
        run = run + jnp.sum(blk, axis=0, keepdims=True)
    c1 = jnp.concatenate(c1_chunks, axis=0)
    counts = run

    pc = jnp.ceil(counts / B) * B
    e_lt = (lax.broadcasted_iota(jnp.int32, (E, E), 0)
            < lax.broadcasted_iota(jnp.int32, (E, E), 1)).astype(jnp.float32)
    poff = lax.dot_general(pc, e_lt, (((1,), (0,)), ((), ())),
                           precision=lax.Precision.HIGHEST,
                           preferred_element_type=jnp.float32)

    d1 = jnp.sum(ind1 * (c1 + poff), axis=1, keepdims=True)
    d2 = jnp.sum(ind2 * (c1 + poff), axis=1, keepdims=True)
    ones16 = jnp.ones((1, 16), jnp.float32)
    v1_ref[...] = m1 * ones16
    v2_ref[...] = m2 * ones16
    d1_ref[...] = d1.astype(jnp.int32)
    d2_ref[...] = d2.astype(jnp.int32)

    eye = (lax.broadcasted_iota(jnp.int32, (E, E), 0)
           == lax.broadcasted_iota(jnp.int32, (E, E), 1)).astype(jnp.float32)
    poff_col = lax.dot_general(eye, poff, (((0,), (1,)), ((), ())),
                               precision=lax.Precision.HIGHEST,
                               preferred_element_type=jnp.float32)
    cnt_col = lax.dot_general(eye, counts, (((0,), (1,)), ((), ())),
                              precision=lax.Precision.HIGHEST,
                              preferred_element_type=jnp.float32)
    bstart = (lax.broadcasted_iota(jnp.int32, (E, NBLK_PAD), 1)
              .astype(jnp.float32) * B)
    cmp = (poff_col <= bstart).astype(jnp.int32)
    be = jnp.sum(cmp, axis=0, keepdims=True) - 1
    live = jnp.logical_and(poff_col <= bstart,
                           bstart < poff_col + cnt_col).astype(jnp.int32)
    valid = jnp.sum(live, axis=0, keepdims=True)
    plan_ref[...] = jnp.concatenate([be, valid], axis=0)


def _run_plan(x, gate_w):
    return pl.pallas_call(
        _plan_body,
        out_shape=(
            jax.ShapeDtypeStruct((T, 16), jnp.float32),
            jax.ShapeDtypeStruct((T, 16), jnp.float32),
            jax.ShapeDtypeStruct((T, 1), jnp.int32),
            jax.ShapeDtypeStruct((T, 1), jnp.int32),
            jax.ShapeDtypeStruct((2, NBLK_PAD), jnp.int32),
        ),
        interpret=_interp,
    )(x, gate_w)


def _gateup_body(plan_ref, xb_ref, wg_ref, wu_ref, h_ref, wgb_ref, wub_ref):
    i = pl.program_id(0)
    new_e = jnp.logical_or(
        i == 0, plan_ref[0, i] != plan_ref[0, jnp.maximum(i - 1, 0)])

    @pl.when(jnp.logical_and(plan_ref[1, i] > 0, new_e))
    def _():
        wgb_ref[...] = wg_ref[0].astype(jnp.bfloat16)
        wub_ref[...] = wu_ref[0].astype(jnp.bfloat16)

    @pl.when(plan_ref[1, i] > 0)
    def _():
        xb = xb_ref[...].astype(jnp.bfloat16)
        g = lax.dot_general(xb, wgb_ref[...], (((1,), (1,)), ((), ())),
                            preferred_element_type=jnp.float32)
        u = lax.dot_general(xb, wub_ref[...], (((1,), (1,)), ((), ())),
                            preferred_element_type=jnp.float32)
        h_ref[...] = ((g * lax.logistic(g)) * u).astype(jnp.bfloat16)


def _down_body(plan_ref, h_ref, w2_ref, y_ref, w2b_ref):
    i = pl.program_id(0)
    new_e = jnp.logical_or(
        i == 0, plan_ref[0, i] != plan_ref[0, jnp.maximum(i - 1, 0)])

    @pl.when(jnp.logical_and(plan_ref[1, i] > 0, new_e))
    def _():
        w2b_ref[...] = w2_ref[0].astype(jnp.bfloat16)

    @pl.when(plan_ref[1, i] > 0)
    def _():
        y_ref[...] = lax.dot_general(h_ref[...], w2b_ref[...],
                                     (((1,), (1,)), ((), ())),
                                     preferred_element_type=jnp.float32)


def _run_ffn(plan, disp, wv1, w2):
    gu_spec = pltpu.PrefetchScalarGridSpec(
        num_scalar_prefetch=1,
        grid=(NBLK,),
        in_specs=[
            pl.BlockSpec((B, D), lambda i, plan: (i, 0)),
            pl.BlockSpec((1, DFF, D), lambda i, plan: (plan[0, i], 0, 0)),
            pl.BlockSpec((1, DFF, D), lambda i, plan: (plan[0, i], 1, 0)),
        ],
        out_specs=pl.BlockSpec((B, DFF), lambda i, plan: (i, 0)),
        scratch_shapes=[pltpu.VMEM((DFF, D), jnp.bfloat16),
                        pltpu.VMEM((DFF, D), jnp.bfloat16)],
    )
    h = pl.pallas_call(
        _gateup_body,
        grid_spec=gu_spec,
        out_shape=jax.ShapeDtypeStruct((NROWS, DFF), jnp.bfloat16),
        compiler_params=pltpu.CompilerParams(
            vmem_limit_bytes=128 * 1024 * 1024),
        interpret=_interp,
    )(plan, disp, wv1, wv1)
    dn_spec = pltpu.PrefetchScalarGridSpec(
        num_scalar_prefetch=1,
        grid=(NBLK,),
        in_specs=[
            pl.BlockSpec((B, DFF), lambda i, plan: (i, 0)),
            pl.BlockSpec((1, D, DFF), lambda i, plan: (plan[0, i], 0, 0)),
        ],
        out_specs=pl.BlockSpec((B, D), lambda i, plan: (i, 0)),
        scratch_shapes=[pltpu.VMEM((D, DFF), jnp.bfloat16)],
    )
    return pl.pallas_call(
        _down_body,
        grid_spec=dn_spec,
        out_shape=jax.ShapeDtypeStruct((NROWS, D), jnp.float32),
        compiler_params=pltpu.CompilerParams(
            vmem_limit_bytes=128 * 1024 * 1024),
        interpret=_interp,
    )(plan, h, w2)


_NC = 2
_NS = 16
NW = _NC * _NS
TPW = T // NW
DCH = 32

_sc_mesh = functools.partial(plsc.VectorSubcoreMesh,
                             core_axis_name="c", subcore_axis_name="s")


def _dispatch_body(x_hbm, d1_hbm, d2_hbm, disp_hbm, i1_m, i2_m, xrows, sem):
    wid = lax.axis_index("s") * _NC + lax.axis_index("c")
    base = wid * TPW
    for s in range(TPW // DCH):
        pltpu.sync_copy(d1_hbm.at[pl.ds(base + s * DCH, DCH)], i1_m.at[s])
        pltpu.sync_copy(d2_hbm.at[pl.ds(base + s * DCH, DCH)], i2_m.at[s])
        pltpu.sync_copy(x_hbm.at[pl.ds(base + s * DCH, DCH)], xrows)
        c1 = pltpu.async_copy(xrows, disp_hbm.at[i1_m.at[s]], sem)
        c2 = pltpu.async_copy(xrows, disp_hbm.at[i2_m.at[s]], sem)
        c1.wait()
        c2.wait()


def _run_dispatch(x, d1, d2):
    return pl.kernel(
        _dispatch_body,
        out_type=jax.ShapeDtypeStruct((NROWS, D), jnp.float32),
        mesh=_sc_mesh(),
        scratch_types=[
            pltpu.VMEM((TPW // DCH, DCH), jnp.int32),
            pltpu.VMEM((TPW // DCH, DCH), jnp.int32),
            pltpu.VMEM((DCH, D), jnp.float32),
            pltpu.SemaphoreType.DMA,
        ],
    )(x, d1, d2)


CT = 16


def _combine_body(y_hbm, d1_hbm, d2_hbm, v1_hbm, v2_hbm, out_hbm,
                  i1_m, i2_m, v1_m, v2_m, ya, yb, ob, sem):
    wid = lax.axis_index("s") * _NC + lax.axis_index("c")
    base = wid * TPW
    nch = TPW // CT
    pltpu.sync_copy(v1_hbm.at[pl.ds(base, TPW)], v1_m)
    pltpu.sync_copy(v2_hbm.at[pl.ds(base, TPW)], v2_m)
    for s in range(nch):
        pltpu.sync_copy(d1_hbm.at[pl.ds(base + s * CT, CT)], i1_m.at[s])
        pltpu.sync_copy(d2_hbm.at[pl.ds(base + s * CT, CT)], i2_m.at[s])
    for s in range(nch):
        ga = pltpu.async_copy(y_hbm.at[i1_m.at[s]], ya, sem)
        gb = pltpu.async_copy(y_hbm.at[i2_m.at[s]], yb, sem)
        ga.wait()
        gb.wait()
        for j in range(CT):
            va = v1_m[s * CT + j]
            vb = v2_m[s * CT + j]

            def q_body(q, _):
                sl = pl.ds(q * 16, 16)
                ob[j, sl] = ya[j, sl] * va + yb[j, sl] * vb
                return 0

            lax.fori_loop(0, D // 16, q_body, 0)
        pltpu.sync_copy(ob, out_hbm.at[pl.ds(base + s * CT, CT)])


def _run_combine(y, d1, d2, v1, v2):
    return pl.kernel(
        _combine_body,
        out_type=jax.ShapeDtypeStruct((T, D), jnp.float32),
        mesh=_sc_mesh(),
        scratch_types=[
            pltpu.VMEM((TPW // CT, CT), jnp.int32),
            pltpu.VMEM((TPW // CT, CT), jnp.int32),
            pltpu.VMEM((TPW, 16), jnp.float32),
            pltpu.VMEM((TPW, 16), jnp.float32),
            pltpu.VMEM((CT, D), jnp.float32),
            pltpu.VMEM((CT, D), jnp.float32),
            pltpu.VMEM((CT, D), jnp.float32),
            pltpu.SemaphoreType.DMA,
        ],
    )(y, d1, d2, v1, v2)


def kernel(x, gate_w, wv1, w2):
    v1, v2, d1, d2, plan = _run_plan(x, gate_w)
    d1 = d1.reshape(T)
    d2 = d2.reshape(T)

    disp = _run_dispatch(x, d1, d2)
    y = _run_ffn(plan, disp, wv1, w2)
    out = _run_combine(y, d1, d2, v1, v2)
    return out.reshape(x.shape)


# speedup vs baseline: 1.0511x; 1.0511x over previous
import functools

import jax
import jax.numpy as jnp
from jax import lax
from jax.experimental import pallas as pl
from jax.experimental.pallas import tpu as pltpu
from jax.experimental.pallas import tpu_sc as plsc

T = 2048
D = 2048
DFF = 1408
E = 8
K = 2

B = 256
NBLK = (T * K) // B + E - 1
NBLK_PAD = 32
NROWS = NBLK * B

_interp = False


def _plan_body(x_ref, gw_ref, v1_ref, v2_ref, d1_ref, d2_ref, plan_ref):
    x = x_ref[...]
    gw = gw_ref[...]
    logits = lax.dot_general(x, gw, (((1,), (1,)), ((), ())),
                             preferred_element_type=jnp.float32)
    m = jnp.max(logits, axis=1, keepdims=True)
    ex = jnp.exp(logits - m)
    p = ex / jnp.sum(ex, axis=1, keepdims=True)

    eio = lax.broadcasted_iota(jnp.int32, (T, E), 1)
    m1 = jnp.max(p, axis=1, keepdims=True)
    i1 = jnp.min(jnp.where(p >= m1, eio, E), axis=1, keepdims=True)
    p2 = jnp.where(eio == i1, -1.0, p)
    m2 = jnp.max(p2, axis=1, keepdims=True)
    i2 = jnp.min(jnp.where(p2 >= m2, eio, E), axis=1, keepdims=True)

    ind1 = (eio == i1).astype(jnp.float32)
    ind2 = (eio == i2).astype(jnp.float32)
    ind12 = ind1 + ind2

    CH = 128
    tri = (lax.broadcasted_iota(jnp.int32, (CH, CH), 0)
           > lax.broadcasted_iota(jnp.int32, (CH, CH), 1)).astype(jnp.float32)
    run = jnp.zeros((1, E), jnp.float32)
    c1_chunks = []
    for i in range(T // CH):
        blk = lax.slice(ind12, (i * CH, 0), ((i + 1) * CH, E))
        c1_chunks.append(lax.dot_general(tri, blk, (((1,), (0,)), ((), ())),
                                         precision=lax.Precision.HIGHEST,
                                         preferred_element_type=jnp.float32)
                         + run)
        run = run + jnp.sum(blk, axis=0, keepdims=True)
    c1 = jnp.concatenate(c1_chunks, axis=0)
    counts = run

    pc = jnp.ceil(counts / B) * B
    e_lt = (lax.broadcasted_iota(jnp.int32, (E, E), 0)
            < lax.broadcasted_iota(jnp.int32, (E, E), 1)).astype(jnp.float32)
    poff = lax.dot_general(pc, e_lt, (((1,), (0,)), ((), ())),
                           precision=lax.Precision.HIGHEST,
                           preferred_element_type=jnp.float32)

    d1 = jnp.sum(ind1 * (c1 + poff), axis=1, keepdims=True)
    d2 = jnp.sum(ind2 * (c1 + poff), axis=1, keepdims=True)
    ones16 = jnp.ones((1, 16), jnp.float32)
    v1_ref[...] = m1 * ones16
    v2_ref[...] = m2 * ones16
    d1_ref[...] = d1.astype(jnp.int32)
    d2_ref[...] = d2.astype(jnp.int32)

    eye = (lax.broadcasted_iota(jnp.int32, (E, E), 0)
           == lax.broadcasted_iota(jnp.int32, (E, E), 1)).astype(jnp.float32)
    poff_col = lax.dot_general(eye, poff, (((0,), (1,)), ((), ())),
                               precision=lax.Precision.HIGHEST,
                               preferred_element_type=jnp.float32)
    cnt_col = lax.dot_general(eye, counts, (((0,), (1,)), ((), ())),
                              precision=lax.Precision.HIGHEST,
                              preferred_element_type=jnp.float32)
    bstart = (lax.broadcasted_iota(jnp.int32, (E, NBLK_PAD), 1)
              .astype(jnp.float32) * B)
    cmp = (poff_col <= bstart).astype(jnp.int32)
    be = jnp.sum(cmp, axis=0, keepdims=True) - 1
    live = jnp.logical_and(poff_col <= bstart,
                           bstart < poff_col + cnt_col).astype(jnp.int32)
    valid = jnp.sum(live, axis=0, keepdims=True)
    plan_ref[...] = jnp.concatenate([be, valid], axis=0)


def _run_plan(x, gate_w):
    return pl.pallas_call(
        _plan_body,
        out_shape=(
            jax.ShapeDtypeStruct((T, 16), jnp.float32),
            jax.ShapeDtypeStruct((T, 16), jnp.float32),
            jax.ShapeDtypeStruct((T, 1), jnp.int32),
            jax.ShapeDtypeStruct((T, 1), jnp.int32),
            jax.ShapeDtypeStruct((2, NBLK_PAD), jnp.int32),
        ),
        interpret=_interp,
    )(x, gate_w)


def _gateup_body(plan_ref, xb_ref, wv_ref, h_ref):
    i = pl.program_id(0)

    @pl.when(plan_ref[1, i] > 0)
    def _():
        xb = xb_ref[...].astype(jnp.bfloat16)
        wv = wv_ref[0].astype(jnp.bfloat16)
        gu = lax.dot_general(xb, wv, (((1,), (1,)), ((), ())),
                             preferred_element_type=jnp.float32)
        g = gu[:, :DFF]
        u = gu[:, DFF:]
        h_ref[...] = ((g * lax.logistic(g)) * u).astype(jnp.bfloat16)


def _down_body(plan_ref, h_ref, w2_ref, y_ref):
    i = pl.program_id(0)

    @pl.when(plan_ref[1, i] > 0)
    def _():
        w2 = w2_ref[0].astype(jnp.bfloat16)
        y_ref[...] = lax.dot_general(h_ref[...], w2, (((1,), (1,)), ((), ())),
                                     preferred_element_type=jnp.float32)


def _run_ffn(plan, disp, wv1, w2):
    gu_spec = pltpu.PrefetchScalarGridSpec(
        num_scalar_prefetch=1,
        grid=(NBLK,),
        in_specs=[
            pl.BlockSpec((B, D), lambda i, plan: (i, 0)),
            pl.BlockSpec((1, 2 * DFF, D), lambda i, plan: (plan[0, i], 0, 0)),
        ],
        out_specs=pl.BlockSpec((B, DFF), lambda i, plan: (i, 0)),
    )
    h = pl.pallas_call(
        _gateup_body,
        grid_spec=gu_spec,
        out_shape=jax.ShapeDtypeStruct((NROWS, DFF), jnp.bfloat16),
        compiler_params=pltpu.CompilerParams(
            vmem_limit_bytes=128 * 1024 * 1024),
        interpret=_interp,
    )(plan, disp, wv1)
    dn_spec = pltpu.PrefetchScalarGridSpec(
        num_scalar_prefetch=1,
        grid=(NBLK,),
        in_specs=[
            pl.BlockSpec((B, DFF), lambda i, plan: (i, 0)),
            pl.BlockSpec((1, D, DFF), lambda i, plan: (plan[0, i], 0, 0)),
        ],
        out_specs=pl.BlockSpec((B, D), lambda i, plan: (i, 0)),
    )
    return pl.pallas_call(
        _down_body,
        grid_spec=dn_spec,
        out_shape=jax.ShapeDtypeStruct((NROWS, D), jnp.float32),
        compiler_params=pltpu.CompilerParams(
            vmem_limit_bytes=128 * 1024 * 1024),
        interpret=_interp,
    )(plan, h, w2)


_NC = 2
_NS = 16
NW = _NC * _NS
TPW = T // NW
DCH = 32

_sc_mesh = functools.partial(plsc.VectorSubcoreMesh,
                             core_axis_name="c", subcore_axis_name="s")


def _dispatch_body(x_hbm, d1_hbm, d2_hbm, disp_hbm, i1_m, i2_m, xrows, sem):
    wid = lax.axis_index("s") * _NC + lax.axis_index("c")
    base = wid * TPW
    for s in range(TPW // DCH):
        pltpu.sync_copy(d1_hbm.at[pl.ds(base + s * DCH, DCH)], i1_m.at[s])
        pltpu.sync_copy(d2_hbm.at[pl.ds(base + s * DCH, DCH)], i2_m.at[s])
        pltpu.sync_copy(x_hbm.at[pl.ds(base + s * DCH, DCH)], xrows)
        c1 = pltpu.async_copy(xrows, disp_hbm.at[i1_m.at[s]], sem)
        c2 = pltpu.async_copy(xrows, disp_hbm.at[i2_m.at[s]], sem)
        c1.wait()
        c2.wait()


def _run_dispatch(x, d1, d2):
    return pl.kernel(
        _dispatch_body,
        out_type=jax.ShapeDtypeStruct((NROWS, D), jnp.float32),
        mesh=_sc_mesh(),
        scratch_types=[
            pltpu.VMEM((TPW // DCH, DCH), jnp.int32),
            pltpu.VMEM((TPW // DCH, DCH), jnp.int32),
            pltpu.VMEM((DCH, D), jnp.float32),
            pltpu.SemaphoreType.DMA,
        ],
    )(x, d1, d2)


CT = 16


def _combine_body(y_hbm, d1_hbm, d2_hbm, v1_hbm, v2_hbm, out_hbm,
                  i1_m, i2_m, v1_m, v2_m, ya, yb, ob, sem):
    wid = lax.axis_index("s") * _NC + lax.axis_index("c")
    base = wid * TPW
    nch = TPW // CT
    pltpu.sync_copy(v1_hbm.at[pl.ds(base, TPW)], v1_m)
    pltpu.sync_copy(v2_hbm.at[pl.ds(base, TPW)], v2_m)
    for s in range(nch):
        pltpu.sync_copy(d1_hbm.at[pl.ds(base + s * CT, CT)], i1_m.at[s])
        pltpu.sync_copy(d2_hbm.at[pl.ds(base + s * CT, CT)], i2_m.at[s])
    for s in range(nch):
        ga = pltpu.async_copy(y_hbm.at[i1_m.at[s]], ya, sem)
        gb = pltpu.async_copy(y_hbm.at[i2_m.at[s]], yb, sem)
        ga.wait()
        gb.wait()
        for j in range(CT):
            va = v1_m[s * CT + j]
            vb = v2_m[s * CT + j]

            def q_body(q, _):
                sl = pl.ds(q * 16, 16)
                ob[j, sl] = ya[j, sl] * va + yb[j, sl] * vb
                return 0

            lax.fori_loop(0, D // 16, q_body, 0)
        pltpu.sync_copy(ob, out_hbm.at[pl.ds(base + s * CT, CT)])


def _run_combine(y, d1, d2, v1, v2):
    return pl.kernel(
        _combine_body,
        out_type=jax.ShapeDtypeStruct((T, D), jnp.float32),
        mesh=_sc_mesh(),
        scratch_types=[
            pltpu.VMEM((TPW // CT, CT), jnp.int32),
            pltpu.VMEM((TPW // CT, CT), jnp.int32),
            pltpu.VMEM((TPW, 16), jnp.float32),
            pltpu.VMEM((TPW, 16), jnp.float32),
            pltpu.VMEM((CT, D), jnp.float32),
            pltpu.VMEM((CT, D), jnp.float32),
            pltpu.VMEM((CT, D), jnp.float32),
            pltpu.SemaphoreType.DMA,
        ],
    )(y, d1, d2, v1, v2)


def kernel(x, gate_w, wv1, w2):
    v1, v2, d1, d2, plan = _run_plan(x, gate_w)
    d1 = d1.reshape(T)
    d2 = d2.reshape(T)

    disp = _run_dispatch(x, d1, d2)
    y = _run_ffn(plan, disp, wv1, w2)
    out = _run_combine(y, d1, d2, v1, v2)
    return out.reshape(x.shape)


# speedup vs baseline: 1.0759x; 1.0237x over previous
import functools

import jax
import jax.numpy as jnp
from jax import lax
from jax.experimental import pallas as pl
from jax.experimental.pallas import tpu as pltpu
from jax.experimental.pallas import tpu_sc as plsc

T = 2048
D = 2048
DFF = 1408
E = 8
K = 2

B = 512
NBLK = (T * K) // B + E - 1
NBLK_PAD = 32
NROWS = NBLK * B

_interp = False


def _plan_body(x_ref, gw_ref, v1_ref, v2_ref, d1_ref, d2_ref, plan_ref):
    x = x_ref[...]
    gw = gw_ref[...]
    logits = lax.dot_general(x, gw, (((1,), (1,)), ((), ())),
                             preferred_element_type=jnp.float32)
    m = jnp.max(logits, axis=1, keepdims=True)
    ex = jnp.exp(logits - m)
    p = ex / jnp.sum(ex, axis=1, keepdims=True)

    eio = lax.broadcasted_iota(jnp.int32, (T, E), 1)
    m1 = jnp.max(p, axis=1, keepdims=True)
    i1 = jnp.min(jnp.where(p >= m1, eio, E), axis=1, keepdims=True)
    p2 = jnp.where(eio == i1, -1.0, p)
    m2 = jnp.max(p2, axis=1, keepdims=True)
    i2 = jnp.min(jnp.where(p2 >= m2, eio, E), axis=1, keepdims=True)

    ind1 = (eio == i1).astype(jnp.float32)
    ind2 = (eio == i2).astype(jnp.float32)
    ind12 = ind1 + ind2

    CH = 128
    tri = (lax.broadcasted_iota(jnp.int32, (CH, CH), 0)
           > lax.broadcasted_iota(jnp.int32, (CH, CH), 1)).astype(jnp.float32)
    run = jnp.zeros((1, E), jnp.float32)
    c1_chunks = []
    for i in range(T // CH):
        blk = lax.slice(ind12, (i * CH, 0), ((i + 1) * CH, E))
        c1_chunks.append(lax.dot_general(tri, blk, (((1,), (0,)), ((), ())),
                                         precision=lax.Precision.HIGHEST,
                                         preferred_element_type=jnp.float32)
                         + run)
        run = run + jnp.sum(blk, axis=0, keepdims=True)
    c1 = jnp.concatenate(c1_chunks, axis=0)
    counts = run

    pc = jnp.ceil(counts / B) * B
    e_lt = (lax.broadcasted_iota(jnp.int32, (E, E), 0)
            < lax.broadcasted_iota(jnp.int32, (E, E), 1)).astype(jnp.float32)
    poff = lax.dot_general(pc, e_lt, (((1,), (0,)), ((), ())),
                           precision=lax.Precision.HIGHEST,
                           preferred_element_type=jnp.float32)

    d1 = jnp.sum(ind1 * (c1 + poff), axis=1, keepdims=True)
    d2 = jnp.sum(ind2 * (c1 + poff), axis=1, keepdims=True)
    ones16 = jnp.ones((1, 16), jnp.float32)
    v1_ref[...] = m1 * ones16
    v2_ref[...] = m2 * ones16
    d1_ref[...] = d1.astype(jnp.int32)
    d2_ref[...] = d2.astype(jnp.int32)

    eye = (lax.broadcasted_iota(jnp.int32, (E, E), 0)
           == lax.broadcasted_iota(jnp.int32, (E, E), 1)).astype(jnp.float32)
    poff_col = lax.dot_general(eye, poff, (((0,), (1,)), ((), ())),
                               precision=lax.Precision.HIGHEST,
                               preferred_element_type=jnp.float32)
    cnt_col = lax.dot_general(eye, counts, (((0,), (1,)), ((), ())),
                              precision=lax.Precision.HIGHEST,
                              preferred_element_type=jnp.float32)
    bstart = (lax.broadcasted_iota(jnp.int32, (E, NBLK_PAD), 1)
              .astype(jnp.float32) * B)
    cmp = (poff_col <= bstart).astype(jnp.int32)
    be = jnp.sum(cmp, axis=0, keepdims=True) - 1
    live = jnp.logical_and(poff_col <= bstart,
                           bstart < poff_col + cnt_col).astype(jnp.int32)
    valid = jnp.sum(live, axis=0, keepdims=True)
    plan_ref[...] = jnp.concatenate([be, valid], axis=0)


def _run_plan(x, gate_w):
    return pl.pallas_call(
        _plan_body,
        out_shape=(
            jax.ShapeDtypeStruct((T, 16), jnp.float32),
            jax.ShapeDtypeStruct((T, 16), jnp.float32),
            jax.ShapeDtypeStruct((T, 1), jnp.int32),
            jax.ShapeDtypeStruct((T, 1), jnp.int32),
            jax.ShapeDtypeStruct((2, NBLK_PAD), jnp.int32),
        ),
        interpret=_interp,
    )(x, gate_w)


def _gateup_body(plan_ref, xb_ref, wv_ref, h_ref):
    i = pl.program_id(0)

    @pl.when(plan_ref[1, i] > 0)
    def _():
        xb = xb_ref[...].astype(jnp.bfloat16)
        wv = wv_ref[0].astype(jnp.bfloat16)
        gu = lax.dot_general(xb, wv, (((1,), (1,)), ((), ())),
                             preferred_element_type=jnp.float32)
        g = gu[:, :DFF]
        u = gu[:, DFF:]
        h_ref[...] = ((g * lax.logistic(g)) * u).astype(jnp.bfloat16)


def _down_body(plan_ref, h_ref, w2_ref, y_ref):
    i = pl.program_id(0)

    @pl.when(plan_ref[1, i] > 0)
    def _():
        w2 = w2_ref[0].astype(jnp.bfloat16)
        y_ref[...] = lax.dot_general(h_ref[...], w2, (((1,), (1,)), ((), ())),
                                     preferred_element_type=jnp.float32)


def _run_ffn(plan, disp, wv1, w2):
    gu_spec = pltpu.PrefetchScalarGridSpec(
        num_scalar_prefetch=1,
        grid=(NBLK,),
        in_specs=[
            pl.BlockSpec((B, D), lambda i, plan: (i, 0)),
            pl.BlockSpec((1, 2 * DFF, D), lambda i, plan: (plan[0, i], 0, 0)),
        ],
        out_specs=pl.BlockSpec((B, DFF), lambda i, plan: (i, 0)),
    )
    h = pl.pallas_call(
        _gateup_body,
        grid_spec=gu_spec,
        out_shape=jax.ShapeDtypeStruct((NROWS, DFF), jnp.bfloat16),
        compiler_params=pltpu.CompilerParams(
            vmem_limit_bytes=128 * 1024 * 1024),
        interpret=_interp,
    )(plan, disp, wv1)
    dn_spec = pltpu.PrefetchScalarGridSpec(
        num_scalar_prefetch=1,
        grid=(NBLK,),
        in_specs=[
            pl.BlockSpec((B, DFF), lambda i, plan: (i, 0)),
            pl.BlockSpec((1, D, DFF), lambda i, plan: (plan[0, i], 0, 0)),
        ],
        out_specs=pl.BlockSpec((B, D), lambda i, plan: (i, 0)),
    )
    return pl.pallas_call(
        _down_body,
        grid_spec=dn_spec,
        out_shape=jax.ShapeDtypeStruct((NROWS, D), jnp.float32),
        compiler_params=pltpu.CompilerParams(
            vmem_limit_bytes=128 * 1024 * 1024),
        interpret=_interp,
    )(plan, h, w2)


_NC = 2
_NS = 16
NW = _NC * _NS
TPW = T // NW
DCH = 32

_sc_mesh = functools.partial(plsc.VectorSubcoreMesh,
                             core_axis_name="c", subcore_axis_name="s")


def _dispatch_body(x_hbm, d1_hbm, d2_hbm, disp_hbm, i1_m, i2_m, xrows, sem):
    wid = lax.axis_index("s") * _NC + lax.axis_index("c")
    base = wid * TPW
    for s in range(TPW // DCH):
        pltpu.sync_copy(d1_hbm.at[pl.ds(base + s * DCH, DCH)], i1_m.at[s])
        pltpu.sync_copy(d2_hbm.at[pl.ds(base + s * DCH, DCH)], i2_m.at[s])
        pltpu.sync_copy(x_hbm.at[pl.ds(base + s * DCH, DCH)], xrows)
        c1 = pltpu.async_copy(xrows, disp_hbm.at[i1_m.at[s]], sem)
        c2 = pltpu.async_copy(xrows, disp_hbm.at[i2_m.at[s]], sem)
        c1.wait()
        c2.wait()


def _run_dispatch(x, d1, d2):
    return pl.kernel(
        _dispatch_body,
        out_type=jax.ShapeDtypeStruct((NROWS, D), jnp.float32),
        mesh=_sc_mesh(),
        scratch_types=[
            pltpu.VMEM((TPW // DCH, DCH), jnp.int32),
            pltpu.VMEM((TPW // DCH, DCH), jnp.int32),
            pltpu.VMEM((DCH, D), jnp.float32),
            pltpu.SemaphoreType.DMA,
        ],
    )(x, d1, d2)


CT = 16


def _combine_body(y_hbm, d1_hbm, d2_hbm, v1_hbm, v2_hbm, out_hbm,
                  i1_m, i2_m, v1_m, v2_m, ya, yb, ob, sem):
    wid = lax.axis_index("s") * _NC + lax.axis_index("c")
    base = wid * TPW
    nch = TPW // CT
    pltpu.sync_copy(v1_hbm.at[pl.ds(base, TPW)], v1_m)
    pltpu.sync_copy(v2_hbm.at[pl.ds(base, TPW)], v2_m)
    for s in range(nch):
        pltpu.sync_copy(d1_hbm.at[pl.ds(base + s * CT, CT)], i1_m.at[s])
        pltpu.sync_copy(d2_hbm.at[pl.ds(base + s * CT, CT)], i2_m.at[s])
    for s in range(nch):
        ga = pltpu.async_copy(y_hbm.at[i1_m.at[s]], ya, sem)
        gb = pltpu.async_copy(y_hbm.at[i2_m.at[s]], yb, sem)
        ga.wait()
        gb.wait()
        for j in range(CT):
            va = v1_m[s * CT + j]
            vb = v2_m[s * CT + j]

            def q_body(q, _):
                sl = pl.ds(q * 16, 16)
                ob[j, sl] = ya[j, sl] * va + yb[j, sl] * vb
                return 0

            lax.fori_loop(0, D // 16, q_body, 0)
        pltpu.sync_copy(ob, out_hbm.at[pl.ds(base + s * CT, CT)])


def _run_combine(y, d1, d2, v1, v2):
    return pl.kernel(
        _combine_body,
        out_type=jax.ShapeDtypeStruct((T, D), jnp.float32),
        mesh=_sc_mesh(),
        scratch_types=[
            pltpu.VMEM((TPW // CT, CT), jnp.int32),
            pltpu.VMEM((TPW // CT, CT), jnp.int32),
            pltpu.VMEM((TPW, 16), jnp.float32),
            pltpu.VMEM((TPW, 16), jnp.float32),
            pltpu.VMEM((CT, D), jnp.float32),
            pltpu.VMEM((CT, D), jnp.float32),
            pltpu.VMEM((CT, D), jnp.float32),
            pltpu.SemaphoreType.DMA,
        ],
    )(y, d1, d2, v1, v2)


def kernel(x, gate_w, wv1, w2):
    v1, v2, d1, d2, plan = _run_plan(x, gate_w)
    d1 = d1.reshape(T)
    d2 = d2.reshape(T)

    disp = _run_dispatch(x, d1, d2)
    y = _run_ffn(plan, disp, wv1, w2)
    out = _run_combine(y, d1, d2, v1, v2)
    return out.reshape(x.shape)


# speedup vs baseline: 1.1688x; 1.0863x over previous
import functools

import jax
import jax.numpy as jnp
from jax import lax
from jax.experimental import pallas as pl
from jax.experimental.pallas import tpu as pltpu
from jax.experimental.pallas import tpu_sc as plsc

T = 2048
D = 2048
DFF = 1408
E = 8
K = 2

B = 512
NBLK = (T * K) // B + E - 1
NBLK_PAD = 32
NROWS = NBLK * B

_interp = False


def _plan_body(x_ref, gw_ref, v1_ref, v2_ref, d1_ref, d2_ref, plan_ref):
    x = x_ref[...]
    gw = gw_ref[...]
    logits = lax.dot_general(x, gw, (((1,), (1,)), ((), ())),
                             preferred_element_type=jnp.float32)
    m = jnp.max(logits, axis=1, keepdims=True)
    ex = jnp.exp(logits - m)
    p = ex / jnp.sum(ex, axis=1, keepdims=True)

    eio = lax.broadcasted_iota(jnp.int32, (T, E), 1)
    m1 = jnp.max(p, axis=1, keepdims=True)
    i1 = jnp.min(jnp.where(p >= m1, eio, E), axis=1, keepdims=True)
    p2 = jnp.where(eio == i1, -1.0, p)
    m2 = jnp.max(p2, axis=1, keepdims=True)
    i2 = jnp.min(jnp.where(p2 >= m2, eio, E), axis=1, keepdims=True)

    ind1 = (eio == i1).astype(jnp.float32)
    ind2 = (eio == i2).astype(jnp.float32)
    ind12 = ind1 + ind2

    CH = 128
    tri = (lax.broadcasted_iota(jnp.int32, (CH, CH), 0)
           > lax.broadcasted_iota(jnp.int32, (CH, CH), 1)).astype(jnp.float32)
    run = jnp.zeros((1, E), jnp.float32)
    c1_chunks = []
    for i in range(T // CH):
        blk = lax.slice(ind12, (i * CH, 0), ((i + 1) * CH, E))
        c1_chunks.append(lax.dot_general(tri, blk, (((1,), (0,)), ((), ())),
                                         precision=lax.Precision.HIGHEST,
                                         preferred_element_type=jnp.float32)
                         + run)
        run = run + jnp.sum(blk, axis=0, keepdims=True)
    c1 = jnp.concatenate(c1_chunks, axis=0)
    counts = run

    pc = jnp.ceil(counts / B) * B
    e_lt = (lax.broadcasted_iota(jnp.int32, (E, E), 0)
            < lax.broadcasted_iota(jnp.int32, (E, E), 1)).astype(jnp.float32)
    poff = lax.dot_general(pc, e_lt, (((1,), (0,)), ((), ())),
                           precision=lax.Precision.HIGHEST,
                           preferred_element_type=jnp.float32)

    d1 = jnp.sum(ind1 * (c1 + poff), axis=1, keepdims=True)
    d2 = jnp.sum(ind2 * (c1 + poff), axis=1, keepdims=True)
    ones16 = jnp.ones((1, 16), jnp.float32)
    v1_ref[...] = m1 * ones16
    v2_ref[...] = m2 * ones16
    d1_ref[...] = d1.astype(jnp.int32)
    d2_ref[...] = d2.astype(jnp.int32)

    eye = (lax.broadcasted_iota(jnp.int32, (E, E), 0)
           == lax.broadcasted_iota(jnp.int32, (E, E), 1)).astype(jnp.float32)
    poff_col = lax.dot_general(eye, poff, (((0,), (1,)), ((), ())),
                               precision=lax.Precision.HIGHEST,
                               preferred_element_type=jnp.float32)
    cnt_col = lax.dot_general(eye, counts, (((0,), (1,)), ((), ())),
                              precision=lax.Precision.HIGHEST,
                              preferred_element_type=jnp.float32)
    bstart = (lax.broadcasted_iota(jnp.int32, (E, NBLK_PAD), 1)
              .astype(jnp.float32) * B)
    cmp = (poff_col <= bstart).astype(jnp.int32)
    be = jnp.sum(cmp, axis=0, keepdims=True) - 1
    live = jnp.logical_and(poff_col <= bstart,
                           bstart < poff_col + cnt_col).astype(jnp.int32)
    valid = jnp.sum(live, axis=0, keepdims=True)
    plan_ref[...] = jnp.concatenate([be, valid], axis=0)


def _run_plan(x, gate_w):
    return pl.pallas_call(
        _plan_body,
        out_shape=(
            jax.ShapeDtypeStruct((T, 16), jnp.float32),
            jax.ShapeDtypeStruct((T, 16), jnp.float32),
            jax.ShapeDtypeStruct((T, 1), jnp.int32),
            jax.ShapeDtypeStruct((T, 1), jnp.int32),
            jax.ShapeDtypeStruct((2, NBLK_PAD), jnp.int32),
        ),
        interpret=_interp,
    )(x, gate_w)


def _gateup_body(plan_ref, xb_ref, wv_ref, h_ref):
    i = pl.program_id(0)

    @pl.when(plan_ref[1, i] > 0)
    def _():
        xb = xb_ref[...].astype(jnp.bfloat16)
        wv = wv_ref[0].astype(jnp.bfloat16)
        gu = lax.dot_general(xb, wv, (((1,), (1,)), ((), ())),
                             preferred_element_type=jnp.float32)
        g = gu[:, :DFF]
        u = gu[:, DFF:]
        h_ref[...] = ((g * lax.logistic(g)) * u).astype(jnp.bfloat16)


def _down_body(plan_ref, h_ref, w2_ref, y_ref):
    i = pl.program_id(0)

    @pl.when(plan_ref[1, i] > 0)
    def _():
        w2 = w2_ref[0].astype(jnp.bfloat16)
        y_ref[...] = lax.dot_general(h_ref[...], w2, (((1,), (1,)), ((), ())),
                                     preferred_element_type=jnp.float32)


def _run_ffn(plan, disp, wv1, w2):
    gu_spec = pltpu.PrefetchScalarGridSpec(
        num_scalar_prefetch=1,
        grid=(NBLK,),
        in_specs=[
            pl.BlockSpec((B, D), lambda i, plan: (i, 0)),
            pl.BlockSpec((1, 2 * DFF, D), lambda i, plan: (plan[0, i], 0, 0)),
        ],
        out_specs=pl.BlockSpec((B, DFF), lambda i, plan: (i, 0)),
    )
    h = pl.pallas_call(
        _gateup_body,
        grid_spec=gu_spec,
        out_shape=jax.ShapeDtypeStruct((NROWS, DFF), jnp.bfloat16),
        compiler_params=pltpu.CompilerParams(
            vmem_limit_bytes=128 * 1024 * 1024),
        interpret=_interp,
    )(plan, disp, wv1)
    dn_spec = pltpu.PrefetchScalarGridSpec(
        num_scalar_prefetch=1,
        grid=(NBLK,),
        in_specs=[
            pl.BlockSpec((B, DFF), lambda i, plan: (i, 0)),
            pl.BlockSpec((1, D, DFF), lambda i, plan: (plan[0, i], 0, 0)),
        ],
        out_specs=pl.BlockSpec((B, D), lambda i, plan: (i, 0)),
    )
    return pl.pallas_call(
        _down_body,
        grid_spec=dn_spec,
        out_shape=jax.ShapeDtypeStruct((NROWS, D), jnp.float32),
        compiler_params=pltpu.CompilerParams(
            vmem_limit_bytes=128 * 1024 * 1024),
        interpret=_interp,
    )(plan, h, w2)


_NC = 2
_NS = 16
NW = _NC * _NS
TPW = T // NW
DCH = 32

_sc_mesh = functools.partial(plsc.VectorSubcoreMesh,
                             core_axis_name="c", subcore_axis_name="s")


def _dispatch_body(x_hbm, d1_hbm, d2_hbm, disp_hbm, i1_m, i2_m, xrows, sem):
    wid = lax.axis_index("s") * _NC + lax.axis_index("c")
    base = wid * TPW
    for s in range(TPW // DCH):
        pltpu.sync_copy(d1_hbm.at[pl.ds(base + s * DCH, DCH)], i1_m.at[s])
        pltpu.sync_copy(d2_hbm.at[pl.ds(base + s * DCH, DCH)], i2_m.at[s])
        pltpu.sync_copy(x_hbm.at[pl.ds(base + s * DCH, DCH)], xrows)
        c1 = pltpu.async_copy(xrows, disp_hbm.at[i1_m.at[s]], sem)
        c2 = pltpu.async_copy(xrows, disp_hbm.at[i2_m.at[s]], sem)
        c1.wait()
        c2.wait()


def _run_dispatch(x, d1, d2):
    return pl.kernel(
        _dispatch_body,
        out_type=jax.ShapeDtypeStruct((NROWS, D), jnp.float32),
        mesh=_sc_mesh(),
        scratch_types=[
            pltpu.VMEM((TPW // DCH, DCH), jnp.int32),
            pltpu.VMEM((TPW // DCH, DCH), jnp.int32),
            pltpu.VMEM((DCH, D), jnp.float32),
            pltpu.SemaphoreType.DMA,
        ],
    )(x, d1, d2)


CT = 8


def _combine_body(y_hbm, d1_hbm, d2_hbm, v1_hbm, v2_hbm, out_hbm,
                  i1_m, i2_m, v1_m, v2_m,
                  ya0, yb0, ya1, yb1, ob0, ob1,
                  sa0, sb0, sa1, sb1, so0, so1):
    wid = lax.axis_index("s") * _NC + lax.axis_index("c")
    base = wid * TPW
    nch = TPW // CT
    pltpu.sync_copy(v1_hbm.at[pl.ds(base, TPW)], v1_m)
    pltpu.sync_copy(v2_hbm.at[pl.ds(base, TPW)], v2_m)
    for s in range(nch):
        pltpu.sync_copy(d1_hbm.at[pl.ds(base + s * CT, CT)], i1_m.at[s])
        pltpu.sync_copy(d2_hbm.at[pl.ds(base + s * CT, CT)], i2_m.at[s])
    ya = [ya0, ya1]
    yb = [yb0, yb1]
    ob = [ob0, ob1]
    sa = [sa0, sa1]
    sb = [sb0, sb1]
    so = [so0, so1]
    ha = [None, None]
    hb = [None, None]
    ho = [None, None]
    ha[0] = pltpu.async_copy(y_hbm.at[i1_m.at[0]], ya[0], sa[0])
    hb[0] = pltpu.async_copy(y_hbm.at[i2_m.at[0]], yb[0], sb[0])
    for s in range(nch):
        cur = s % 2
        nxt = 1 - cur
        if s + 1 < nch:
            ha[nxt] = pltpu.async_copy(y_hbm.at[i1_m.at[s + 1]], ya[nxt],
                                       sa[nxt])
            hb[nxt] = pltpu.async_copy(y_hbm.at[i2_m.at[s + 1]], yb[nxt],
                                       sb[nxt])
        ha[cur].wait()
        hb[cur].wait()
        if s >= 2 and ho[cur] is not None:
            ho[cur].wait()
        yac, ybc, obc = ya[cur], yb[cur], ob[cur]
        for j in range(CT):
            va = v1_m[s * CT + j]
            vb = v2_m[s * CT + j]

            def q_body(q, _, j=j, yac=yac, ybc=ybc, obc=obc, va=va, vb=vb):
                for r in range(8):
                    sl = pl.ds(q * 128 + r * 16, 16)
                    obc[j, sl] = yac[j, sl] * va + ybc[j, sl] * vb
                return 0

            lax.fori_loop(0, D // 128, q_body, 0)
        ho[cur] = pltpu.async_copy(ob[cur],
                                   out_hbm.at[pl.ds(base + s * CT, CT)],
                                   so[cur])
    ho[0].wait()
    ho[1].wait()


def _run_combine(y, d1, d2, v1, v2):
    return pl.kernel(
        _combine_body,
        out_type=jax.ShapeDtypeStruct((T, D), jnp.float32),
        mesh=_sc_mesh(),
        scratch_types=[
            pltpu.VMEM((TPW // CT, CT), jnp.int32),
            pltpu.VMEM((TPW // CT, CT), jnp.int32),
            pltpu.VMEM((TPW, 16), jnp.float32),
            pltpu.VMEM((TPW, 16), jnp.float32),
            pltpu.VMEM((CT, D), jnp.float32),
            pltpu.VMEM((CT, D), jnp.float32),
            pltpu.VMEM((CT, D), jnp.float32),
            pltpu.VMEM((CT, D), jnp.float32),
            pltpu.VMEM((CT, D), jnp.float32),
            pltpu.VMEM((CT, D), jnp.float32),
            pltpu.SemaphoreType.DMA,
            pltpu.SemaphoreType.DMA,
            pltpu.SemaphoreType.DMA,
            pltpu.SemaphoreType.DMA,
            pltpu.SemaphoreType.DMA,
            pltpu.SemaphoreType.DMA,
        ],
    )(y, d1, d2, v1, v2)


def kernel(x, gate_w, wv1, w2):
    v1, v2, d1, d2, plan = _run_plan(x, gate_w)
    d1 = d1.reshape(T)
    d2 = d2.reshape(T)

    disp = _run_dispatch(x, d1, d2)
    y = _run_ffn(plan, disp, wv1, w2)
    out = _run_combine(y, d1, d2, v1, v2)
    return out.reshape(x.shape)
